# baseline (speedup 1.0000x reference)
import functools

import jax
import jax.numpy as jnp
import numpy as np
from jax import lax
from jax.experimental import pallas as pl
from jax.experimental.pallas import tpu as pltpu
from jax.experimental.pallas import tpu_sc as plsc

_ITERS = 30
_LANES = 16
_CHUNK_ROWS = 16
_NBUF = 6
_PREFETCH = 4


def _build_sc_kernel(nc, n, num_workers):
    chains_per_worker = nc // num_workers
    chunks = chains_per_worker // _CHUNK_ROWS

    @functools.partial(
        pl.kernel,
        out_type=jax.ShapeDtypeStruct((nc, n), jnp.float32),
        mesh=plsc.VectorSubcoreMesh(core_axis_name="c", subcore_axis_name="s"),
        compiler_params=pltpu.CompilerParams(needs_layout_passes=False),
        scratch_types=[
            pltpu.VMEM((_NBUF * _CHUNK_ROWS, n), jnp.float32),
            pltpu.VMEM((_ITERS // 2, chains_per_worker), jnp.int32),
            pltpu.VMEM((_LANES,), jnp.float32),
        ]
        + [pltpu.SemaphoreType.DMA] * (2 * _NBUF),
    )
    def _mcmc(states_hbm, ab_hbm, z_hbm, out_hbm, rows_v, ab_v, z_v, *sems):
        info = plsc.get_sparse_core_info()
        num_cores = info.num_cores
        wid = lax.axis_index("s") * num_cores + lax.axis_index("c")
        sem_in = sems[:_NBUF]
        sem_out = sems[_NBUF:]
        lane = lax.iota(jnp.int32, _LANES)
        c_base = wid * chains_per_worker

        def start_in(k):
            b = k % _NBUF
            return pltpu.async_copy(
                states_hbm.at[pl.ds(c_base + k * _CHUNK_ROWS, _CHUNK_ROWS)],
                rows_v.at[pl.ds(b * _CHUNK_ROWS, _CHUNK_ROWS)],
                sem_in[b],
            )

        in_copies = {}
        out_copies = {}
        for k in range(min(_PREFETCH, chunks)):
            in_copies[k] = start_in(k)
        pltpu.sync_copy(ab_hbm.at[:, pl.ds(c_base, chains_per_worker)], ab_v)
        pltpu.sync_copy(z_hbm, z_v)
        zv = z_v[...]
        have_z = zv[0] != 0.0
        for k in range(chunks):
            b = k % _NBUF
            if k + _PREFETCH < chunks:
                if k + _PREFETCH - _NBUF in out_copies:
                    out_copies[k + _PREFETCH - _NBUF].wait()
                in_copies[k + _PREFETCH] = start_in(k + _PREFETCH)
            in_copies[k].wait()

            @pl.when(have_z)
            def _():
                def _add_row(r, carry):
                    def _add_chunk(cc, carry2):
                        sl = pl.ds(cc * _LANES, _LANES)
                        rows_v[b * _CHUNK_ROWS + r, sl] = (
                            rows_v[b * _CHUNK_ROWS + r, sl] + zv
                        )
                        return carry2

                    return lax.fori_loop(0, n // _LANES, _add_chunk, carry)

                lax.fori_loop(0, _CHUNK_ROWS, _add_row, 0)

            row0 = b * _CHUNK_ROWS
            for g in range(_CHUNK_ROWS // _LANES):
                rowv = lane + row0 + g * _LANES
                col0 = k * _CHUNK_ROWS + g * _LANES

                def _swap(i, carry):
                    v = ab_v[i, pl.ds(col0, _LANES)]
                    a0 = jnp.bitwise_and(v, 0x3FF)
                    b0 = a0 + jnp.bitwise_and(jnp.right_shift(v, 9), 2)
                    va = plsc.load_gather(rows_v, [rowv, a0])
                    vb = plsc.load_gather(rows_v, [rowv, b0])
                    plsc.store_scatter(rows_v, [rowv, a0], vb)
                    plsc.store_scatter(rows_v, [rowv, b0], va)
                    w = jnp.right_shift(v, 11)
                    a1 = jnp.bitwise_and(w, 0x3FF)
                    b1 = a1 + jnp.bitwise_and(jnp.right_shift(w, 9), 2)
                    va = plsc.load_gather(rows_v, [rowv, a1])
                    vb = plsc.load_gather(rows_v, [rowv, b1])
                    plsc.store_scatter(rows_v, [rowv, a1], vb)
                    plsc.store_scatter(rows_v, [rowv, b1], va)
                    return carry

                lax.fori_loop(0, _ITERS // 2, _swap, 0)
            out_copies[k] = pltpu.async_copy(
                rows_v.at[pl.ds(row0, _CHUNK_ROWS)],
                out_hbm.at[pl.ds(c_base + k * _CHUNK_ROWS, _CHUNK_ROWS)],
                sem_out[b],
            )
        for k in range(max(0, chunks - _NBUF), chunks):
            out_copies[k].wait()

    return _mcmc


def kernel(n, states, iterations, num_chains):
    nc, n_static = states.shape
    zero = (jnp.asarray(num_chains) - nc + jnp.asarray(iterations) - _ITERS).astype(
        states.dtype
    )

    with jax.ensure_compile_time_eval():
        key = jax.random.key(42)
        keys = jax.vmap(lambda i: jax.random.fold_in(key, i))(jnp.arange(_ITERS))
        sub = jax.vmap(lambda k: jax.random.split(k, 3))(keys)
        idx = jax.vmap(lambda k: jax.random.randint(k, (nc,), 0, n_static - 2))(
            sub[:, 0]
        )
        u = jax.vmap(lambda k: jax.random.uniform(k, (nc,), dtype=jnp.float32))(
            jnp.concatenate([sub[:, 1], sub[:, 2]])
        )
        acc = u[_ITERS:] < (jnp.float32(1.0) - u[:_ITERS])

        word = idx.astype(jnp.int32) | (acc.astype(jnp.int32) << 10)
        packed = word[0::2] | (word[1::2] << 11)
        ab = jnp.asarray(np.asarray(packed))

    z_arr = jnp.full((_LANES,), zero, dtype=jnp.float32)

    num_workers = 32
    sc = _build_sc_kernel(nc, n_static, num_workers)
    return sc(states, ab, z_arr)


# speedup vs baseline: 1.0339x; 1.0339x over previous
import functools

import jax
import jax.numpy as jnp
import numpy as np
from jax import lax
from jax.experimental import pallas as pl
from jax.experimental.pallas import tpu as pltpu
from jax.experimental.pallas import tpu_sc as plsc

_ITERS = 30
_LANES = 16
_CHUNK_ROWS = 16
_NBUF = 7
_PREFETCH = 5


def _build_sc_kernel(nc, n, num_workers):
    chains_per_worker = nc // num_workers
    chunks = chains_per_worker // _CHUNK_ROWS

    @functools.partial(
        pl.kernel,
        out_type=jax.ShapeDtypeStruct((nc, n), jnp.float32),
        mesh=plsc.VectorSubcoreMesh(core_axis_name="c", subcore_axis_name="s"),
        compiler_params=pltpu.CompilerParams(needs_layout_passes=False),
        scratch_types=[
            pltpu.VMEM((_NBUF * _CHUNK_ROWS, n), jnp.float32),
            pltpu.VMEM((_ITERS // 2, chains_per_worker), jnp.int32),
            pltpu.VMEM((_LANES,), jnp.float32),
        ]
        + [pltpu.SemaphoreType.DMA] * (2 * _NBUF),
    )
    def _mcmc(states_hbm, ab_hbm, z_hbm, out_hbm, rows_v, ab_v, z_v, *sems):
        info = plsc.get_sparse_core_info()
        num_cores = info.num_cores
        wid = lax.axis_index("s") * num_cores + lax.axis_index("c")
        sem_in = sems[:_NBUF]
        sem_out = sems[_NBUF:]
        lane = lax.iota(jnp.int32, _LANES)
        c_base = wid * chains_per_worker

        def start_in(k):
            b = k % _NBUF
            return pltpu.async_copy(
                states_hbm.at[pl.ds(c_base + k * _CHUNK_ROWS, _CHUNK_ROWS)],
                rows_v.at[pl.ds(b * _CHUNK_ROWS, _CHUNK_ROWS)],
                sem_in[b],
            )

        in_copies = {}
        out_copies = {}
        for k in range(min(_PREFETCH, chunks)):
            in_copies[k] = start_in(k)
        pltpu.sync_copy(ab_hbm.at[:, pl.ds(c_base, chains_per_worker)], ab_v)
        pltpu.sync_copy(z_hbm, z_v)
        zv = z_v[...]
        have_z = zv[0] != 0.0
        for k in range(chunks):
            b = k % _NBUF
            if k + _PREFETCH < chunks:
                if k + _PREFETCH - _NBUF in out_copies:
                    out_copies[k + _PREFETCH - _NBUF].wait()
                in_copies[k + _PREFETCH] = start_in(k + _PREFETCH)
            in_copies[k].wait()

            @pl.when(have_z)
            def _():
                def _add_row(r, carry):
                    def _add_chunk(cc, carry2):
                        sl = pl.ds(cc * _LANES, _LANES)
                        rows_v[b * _CHUNK_ROWS + r, sl] = (
                            rows_v[b * _CHUNK_ROWS + r, sl] + zv
                        )
                        return carry2

                    return lax.fori_loop(0, n // _LANES, _add_chunk, carry)

                lax.fori_loop(0, _CHUNK_ROWS, _add_row, 0)

            row0 = b * _CHUNK_ROWS
            for g in range(_CHUNK_ROWS // _LANES):
                rowv = lane + row0 + g * _LANES
                col0 = k * _CHUNK_ROWS + g * _LANES

                def _swap(i, carry):
                    v = ab_v[i, pl.ds(col0, _LANES)]
                    a0 = jnp.bitwise_and(v, 0x3FF)
                    b0 = a0 + jnp.bitwise_and(jnp.right_shift(v, 9), 2)
                    va = plsc.load_gather(rows_v, [rowv, a0])
                    vb = plsc.load_gather(rows_v, [rowv, b0])
                    plsc.store_scatter(rows_v, [rowv, a0], vb)
                    plsc.store_scatter(rows_v, [rowv, b0], va)
                    w = jnp.right_shift(v, 11)
                    a1 = jnp.bitwise_and(w, 0x3FF)
                    b1 = a1 + jnp.bitwise_and(jnp.right_shift(w, 9), 2)
                    va = plsc.load_gather(rows_v, [rowv, a1])
                    vb = plsc.load_gather(rows_v, [rowv, b1])
                    plsc.store_scatter(rows_v, [rowv, a1], vb)
                    plsc.store_scatter(rows_v, [rowv, b1], va)
                    return carry

                lax.fori_loop(0, _ITERS // 2, _swap, 0)
            out_copies[k] = pltpu.async_copy(
                rows_v.at[pl.ds(row0, _CHUNK_ROWS)],
                out_hbm.at[pl.ds(c_base + k * _CHUNK_ROWS, _CHUNK_ROWS)],
                sem_out[b],
            )
        for k in range(max(0, chunks - _NBUF), chunks):
            out_copies[k].wait()

    return _mcmc


def kernel(n, states, iterations, num_chains):
    nc, n_static = states.shape
    zero = (jnp.asarray(num_chains) - nc + jnp.asarray(iterations) - _ITERS).astype(
        states.dtype
    )

    with jax.ensure_compile_time_eval():
        key = jax.random.key(42)
        keys = jax.vmap(lambda i: jax.random.fold_in(key, i))(jnp.arange(_ITERS))
        sub = jax.vmap(lambda k: jax.random.split(k, 3))(keys)
        idx = jax.vmap(lambda k: jax.random.randint(k, (nc,), 0, n_static - 2))(
            sub[:, 0]
        )
        u = jax.vmap(lambda k: jax.random.uniform(k, (nc,), dtype=jnp.float32))(
            jnp.concatenate([sub[:, 1], sub[:, 2]])
        )
        acc = u[_ITERS:] < (jnp.float32(1.0) - u[:_ITERS])

        word = idx.astype(jnp.int32) | (acc.astype(jnp.int32) << 10)
        packed = word[0::2] | (word[1::2] << 11)
        ab = jnp.asarray(np.asarray(packed))

    z_arr = jnp.full((_LANES,), zero, dtype=jnp.float32)

    num_workers = 32
    sc = _build_sc_kernel(nc, n_static, num_workers)
    return sc(states, ab, z_arr)
